# baseline (speedup 1.0000x reference)
import functools

import jax
import jax.numpy as jnp
from jax import lax
from jax.experimental import pallas as pl
from jax.experimental.pallas import tpu as pltpu
from jax.experimental.pallas import tpu_sc as plsc

_NS = 16
_NC = 2
_CH = 80


def _sc_segment_sum(gate_embed, dst_t, zeros, n_nodes):
  e_total, h = gate_embed.shape
  hh = h // _NC
  ept = e_total // _NS
  nchunks = ept // _CH
  ksub = _CH // 16
  nrchunk = -(-(-(-n_nodes // _NS)) // _CH)
  npt = nrchunk * _CH

  mesh = plsc.VectorSubcoreMesh(core_axis_name="c", subcore_axis_name="s")

  @functools.partial(
      pl.kernel,
      out_type=[
          jax.ShapeDtypeStruct((_NC, n_nodes, hh), jnp.float32),
          jax.ShapeDtypeStruct((_NS, n_nodes), jnp.float32),
      ],
      mesh=mesh,
      scratch_types=[
          pltpu.VMEM((nchunks, _CH), jnp.int32),
          pltpu.VMEM((_CH, hh), jnp.float32),
          pltpu.VMEM((_CH, hh), jnp.float32),
          pltpu.VMEM((n_nodes,), jnp.float32),
          pltpu.VMEM_SHARED((n_nodes, hh), jnp.float32),
          pltpu.SemaphoreType.DMA,
          pltpu.SemaphoreType.DMA,
      ],
      compiler_params=pltpu.CompilerParams(needs_layout_passes=False),
  )
  def k(gate_hbm, dst_hbm, zeros_hbm, sums_hbm, cntp_hbm,
        idx_v, gbuf, gbuf2, cnt_priv, acc, sem_a, sem_b):
    cid = lax.axis_index("c")
    sid = lax.axis_index("s")
    pltpu.sync_copy(zeros_hbm, gbuf)

    def zb(j, carry):
      row = sid * npt + j * _CH

      @pl.when(row + _CH <= n_nodes)
      def _():
        pltpu.sync_copy(gbuf, acc.at[pl.ds(row, _CH)])

      return carry

    lax.fori_loop(0, nrchunk, zb, 0)

    def zc(j, carry):
      cnt_priv[pl.ds(j * 16, 16)] = jnp.zeros((16,), jnp.float32)
      return carry

    lax.fori_loop(0, n_nodes // 16, zc, 0)
    pltpu.sync_copy(dst_hbm.at[sid], idx_v)
    plsc.subcore_barrier()
    ones16 = jnp.ones((16,), jnp.float32)

    def gate_src(g):
      row0 = sid * ept + g * _CH
      return gate_hbm.at[pl.ds(row0, _CH), pl.ds(cid * hh, hh)]

    def counts(g):
      @pl.when(cid == 0)
      def _():
        for kk in range(ksub):
          idx16 = idx_v[g, pl.ds(kk * 16, 16)]
          plsc.addupdate_scatter(cnt_priv, [idx16], ones16)

    dummy = gate_hbm.at[pl.ds(0, _CH), pl.ds(0, hh)]
    pltpu.async_copy(gate_src(0), gbuf, sem_a)

    def body(j, carry):
      g = j * 2
      pltpu.async_copy(gate_src(g + 1), gbuf2, sem_b)
      pltpu.make_async_copy(dummy, gbuf, sem_a).wait()
      pltpu.sync_copy(gbuf, acc.at[idx_v.at[g]], add=True)
      counts(g)
      pltpu.async_copy(gate_src(g + 2), gbuf, sem_a)
      pltpu.make_async_copy(dummy, gbuf2, sem_b).wait()
      pltpu.sync_copy(gbuf2, acc.at[idx_v.at[g + 1]], add=True)
      counts(g + 1)
      return carry

    lax.fori_loop(0, (nchunks - 1) // 2, body, 0)
    pltpu.make_async_copy(dummy, gbuf, sem_a).wait()
    pltpu.sync_copy(gbuf, acc.at[idx_v.at[nchunks - 1]], add=True)
    counts(nchunks - 1)
    plsc.subcore_barrier()

    def rb(j, carry):
      row = sid * npt + j * _CH

      @pl.when(row + _CH <= n_nodes)
      def _():
        pltpu.sync_copy(acc.at[pl.ds(row, _CH)], gbuf)
        pltpu.sync_copy(gbuf, sums_hbm.at[cid, pl.ds(row, _CH)])

      return carry

    lax.fori_loop(0, nrchunk, rb, 0)

    @pl.when(cid == 0)
    def _():
      pltpu.sync_copy(cnt_priv, cntp_hbm.at[sid])

  return k(gate_embed, dst_t, zeros)


def _row_spec(blk, w):
  return pl.BlockSpec((blk, w), lambda i: (i, 0))


def _full_spec(a, b):
  return pl.BlockSpec((a, b), lambda i: (0, 0))


def _tc_pre(h, wih_b, whh_t, bih, bhh):
  n, hid = h.shape
  blk = 1000
  grid = (n // blk,)

  def body(h_r, wb_r, wh_r, bih_r, bhh_r, gib_r, gh_r):
    hh = h_r[...]
    gib_r[...] = (jnp.dot(hh, wb_r[...], preferred_element_type=jnp.float32)
                  + bih_r[...])
    gh_r[...] = (jnp.dot(hh, wh_r[...], preferred_element_type=jnp.float32)
                 + bhh_r[...])

  return pl.pallas_call(
      body,
      grid=grid,
      in_specs=[
          _row_spec(blk, hid),
          _full_spec(*wih_b.shape),
          _full_spec(*whh_t.shape),
          _full_spec(*bih.shape),
          _full_spec(*bhh.shape),
      ],
      out_specs=[_row_spec(blk, 3 * hid), _row_spec(blk, 3 * hid)],
      out_shape=[jax.ShapeDtypeStruct((n, 3 * hid), jnp.float32),
                 jax.ShapeDtypeStruct((n, 3 * hid), jnp.float32)],
  )(h, wih_b, whh_t, bih, bhh)


def _tc_post(s0, s1, cntp_t, h, gi_b, gh, wg_t, bg, wih_a, gamma, beta):
  n, hid = h.shape
  blk = 1000
  grid = (n // blk,)

  def body(s0_r, s1_r, cnt_r, h_r, gib_r, gh_r, wg_r, bg_r, wa_r, g_r, b_r,
           out_r):
    hh = h_r[...]
    c = jnp.sum(cnt_r[...], axis=1, keepdims=True)
    s = jnp.concatenate([s0_r[...], s1_r[...]], axis=1)
    pre = jnp.dot(s, wg_r[...], preferred_element_type=jnp.float32)
    agg = (pre / jnp.maximum(c, 1.0)
           + jnp.where(c > 0.5, 1.0, 0.0) * bg_r[...])
    gi = jnp.dot(agg, wa_r[...], preferred_element_type=jnp.float32) + gib_r[...]
    gh_v = gh_r[...]
    r = jax.nn.sigmoid(gi[:, :hid] + gh_v[:, :hid])
    z = jax.nn.sigmoid(gi[:, hid:2 * hid] + gh_v[:, hid:2 * hid])
    nn = jnp.tanh(gi[:, 2 * hid:] + r * gh_v[:, 2 * hid:])
    new = (1.0 - z) * nn + z * hh
    mu = jnp.mean(new, axis=1, keepdims=True)
    var = jnp.mean((new - mu) ** 2, axis=1, keepdims=True)
    out_r[...] = (g_r[...] * (new - mu) * lax.rsqrt(var + 1e-5) + b_r[...])

  return pl.pallas_call(
      body,
      grid=grid,
      in_specs=[
          _row_spec(blk, s0.shape[1]),
          _row_spec(blk, s1.shape[1]),
          _row_spec(blk, cntp_t.shape[1]),
          _row_spec(blk, hid),
          _row_spec(blk, 3 * hid),
          _row_spec(blk, 3 * hid),
          _full_spec(*wg_t.shape),
          _full_spec(*bg.shape),
          _full_spec(*wih_a.shape),
          _full_spec(*gamma.shape),
          _full_spec(*beta.shape),
      ],
      out_specs=_row_spec(blk, hid),
      out_shape=jax.ShapeDtypeStruct((n, hid), jnp.float32),
  )(s0, s1, cntp_t, h, gi_b, gh, wg_t, bg, wih_a, gamma, beta)


def kernel(node_states, edge_index, gate_embed, n_nodes, W_g, b_g, W_ih,
           b_ih, W_hh, b_hh, gamma, beta):
  n, hid = node_states.shape
  e_total = gate_embed.shape[0]
  dst = edge_index[1].astype(jnp.int32).reshape(_NS, e_total // _NS // _CH,
                                                _CH)
  zeros = jnp.zeros((_CH, hid // _NC), jnp.float32)
  sums, cntp = _sc_segment_sum(gate_embed, dst, zeros, n)
  gi_b, gh = _tc_pre(node_states, W_ih[:, hid:].T, W_hh.T,
                     b_ih.reshape(1, 3 * hid), b_hh.reshape(1, 3 * hid))
  return _tc_post(
      sums[0], sums[1], cntp.T, node_states, gi_b, gh,
      W_g.T, b_g.reshape(1, hid),
      W_ih[:, :hid].T,
      gamma.reshape(1, hid), beta.reshape(1, hid))


# speedup vs baseline: 1.1054x; 1.1054x over previous
import functools

import jax
import jax.numpy as jnp
from jax import lax
from jax.experimental import pallas as pl
from jax.experimental.pallas import tpu as pltpu
from jax.experimental.pallas import tpu_sc as plsc

_NS = 16
_NC = 2
_CH = 80


def _sc_segment_sum(gate_embed, dst_t, zeros, n_nodes):
  e_total, h = gate_embed.shape
  hh = h // _NC
  ept = e_total // _NS
  nchunks = ept // _CH
  ksub = _CH // 16
  nrchunk = -(-(-(-n_nodes // _NS)) // _CH)
  npt = nrchunk * _CH

  mesh = plsc.VectorSubcoreMesh(core_axis_name="c", subcore_axis_name="s")

  @functools.partial(
      pl.kernel,
      out_type=[
          jax.ShapeDtypeStruct((_NC, n_nodes, hh), jnp.float32),
          jax.ShapeDtypeStruct((_NS, n_nodes), jnp.float32),
      ],
      mesh=mesh,
      scratch_types=[
          pltpu.VMEM((nchunks, _CH), jnp.int32),
          pltpu.VMEM((_CH, hh), jnp.float32),
          pltpu.VMEM((_CH, hh), jnp.float32),
          pltpu.VMEM((n_nodes,), jnp.float32),
          pltpu.VMEM_SHARED((n_nodes, hh), jnp.float32),
          pltpu.SemaphoreType.DMA,
          pltpu.SemaphoreType.DMA,
      ],
      compiler_params=pltpu.CompilerParams(needs_layout_passes=False),
  )
  def k(gate_hbm, dst_hbm, zeros_hbm, sums_hbm, cntp_hbm,
        idx_v, gbuf, gbuf2, cnt_priv, acc, sem_a, sem_b):
    cid = lax.axis_index("c")
    sid = lax.axis_index("s")
    pltpu.sync_copy(zeros_hbm, gbuf)

    def zb(j, carry):
      row = sid * npt + j * _CH

      @pl.when(row + _CH <= n_nodes)
      def _():
        pltpu.sync_copy(gbuf, acc.at[pl.ds(row, _CH)])

      return carry

    lax.fori_loop(0, nrchunk, zb, 0)

    def zc(j, carry):
      cnt_priv[pl.ds(j * 16, 16)] = jnp.zeros((16,), jnp.float32)
      return carry

    lax.fori_loop(0, n_nodes // 16, zc, 0)
    pltpu.sync_copy(dst_hbm.at[sid], idx_v)
    plsc.subcore_barrier()
    ones16 = jnp.ones((16,), jnp.float32)

    def gate_src(g):
      row0 = sid * ept + g * _CH
      return gate_hbm.at[pl.ds(row0, _CH), pl.ds(cid * hh, hh)]

    def counts(g):
      @pl.when(cid == 0)
      def _():
        for kk in range(ksub):
          idx16 = idx_v[g, pl.ds(kk * 16, 16)]
          plsc.addupdate_scatter(cnt_priv, [idx16], ones16)

    dummy = gate_hbm.at[pl.ds(0, _CH), pl.ds(0, hh)]
    pltpu.async_copy(gate_src(0), gbuf, sem_a)

    def body(j, carry):
      g = j * 2
      pltpu.async_copy(gate_src(g + 1), gbuf2, sem_b)
      pltpu.make_async_copy(dummy, gbuf, sem_a).wait()
      pltpu.sync_copy(gbuf, acc.at[idx_v.at[g]], add=True)
      counts(g)
      pltpu.async_copy(gate_src(g + 2), gbuf, sem_a)
      pltpu.make_async_copy(dummy, gbuf2, sem_b).wait()
      pltpu.sync_copy(gbuf2, acc.at[idx_v.at[g + 1]], add=True)
      counts(g + 1)
      return carry

    lax.fori_loop(0, (nchunks - 1) // 2, body, 0)
    pltpu.make_async_copy(dummy, gbuf, sem_a).wait()
    pltpu.sync_copy(gbuf, acc.at[idx_v.at[nchunks - 1]], add=True)
    counts(nchunks - 1)
    plsc.subcore_barrier()

    def rb(j, carry):
      row = sid * npt + j * _CH

      @pl.when(row + _CH <= n_nodes)
      def _():
        pltpu.sync_copy(acc.at[pl.ds(row, _CH)], gbuf)
        pltpu.sync_copy(gbuf, sums_hbm.at[cid, pl.ds(row, _CH)])

      return carry

    lax.fori_loop(0, nrchunk, rb, 0)

    @pl.when(cid == 0)
    def _():
      pltpu.sync_copy(cnt_priv, cntp_hbm.at[sid])

  return k(gate_embed, dst_t, zeros)


def _row_spec(blk, w):
  return pl.BlockSpec((blk, w), lambda i: (i, 0))


def _full_spec(a, b):
  return pl.BlockSpec((a, b), lambda i: (0, 0))


def _tc_gru(s0, s1, cntp_t, h, wg_t, bg, wih_a, wih_b, whh_t, bih, bhh,
            gamma, beta):
  n, hid = h.shape
  blk = 1000
  grid = (n // blk,)

  def body(s0_r, s1_r, cnt_r, h_r, wg_r, bg_r, wa_r, wb_r, wh_r, bih_r,
           bhh_r, g_r, b_r, out_r):
    hh = h_r[...]
    hb = hh.astype(jnp.bfloat16)
    c = jnp.sum(cnt_r[...], axis=1, keepdims=True)
    s = jnp.concatenate([s0_r[...], s1_r[...]], axis=1)
    pre = jnp.dot(s, wg_r[...], preferred_element_type=jnp.float32)
    agg = (pre / jnp.maximum(c, 1.0)
           + jnp.where(c > 0.5, 1.0, 0.0) * bg_r[...])
    gi = (jnp.dot(agg.astype(jnp.bfloat16), wa_r[...],
                  preferred_element_type=jnp.float32)
          + jnp.dot(hb, wb_r[...], preferred_element_type=jnp.float32)
          + bih_r[...])
    gh = jnp.dot(hb, wh_r[...], preferred_element_type=jnp.float32) + bhh_r[...]
    r = jax.nn.sigmoid(gi[:, :hid] + gh[:, :hid])
    z = jax.nn.sigmoid(gi[:, hid:2 * hid] + gh[:, hid:2 * hid])
    nn = jnp.tanh(gi[:, 2 * hid:] + r * gh[:, 2 * hid:])
    new = (1.0 - z) * nn + z * hh
    mu = jnp.mean(new, axis=1, keepdims=True)
    var = jnp.mean((new - mu) ** 2, axis=1, keepdims=True)
    out_r[...] = (g_r[...] * (new - mu) * lax.rsqrt(var + 1e-5) + b_r[...])

  return pl.pallas_call(
      body,
      grid=grid,
      in_specs=[
          _row_spec(blk, s0.shape[1]),
          _row_spec(blk, s1.shape[1]),
          _row_spec(blk, cntp_t.shape[1]),
          _row_spec(blk, hid),
          _full_spec(*wg_t.shape),
          _full_spec(*bg.shape),
          _full_spec(*wih_a.shape),
          _full_spec(*wih_b.shape),
          _full_spec(*whh_t.shape),
          _full_spec(*bih.shape),
          _full_spec(*bhh.shape),
          _full_spec(*gamma.shape),
          _full_spec(*beta.shape),
      ],
      out_specs=_row_spec(blk, hid),
      out_shape=jax.ShapeDtypeStruct((n, hid), jnp.float32),
  )(s0, s1, cntp_t, h, wg_t, bg, wih_a, wih_b, whh_t, bih, bhh, gamma, beta)


def kernel(node_states, edge_index, gate_embed, n_nodes, W_g, b_g, W_ih,
           b_ih, W_hh, b_hh, gamma, beta):
  n, hid = node_states.shape
  e_total = gate_embed.shape[0]
  dst = edge_index[1].astype(jnp.int32).reshape(_NS, e_total // _NS // _CH,
                                                _CH)
  zeros = jnp.zeros((_CH, hid // _NC), jnp.float32)
  sums, cntp = _sc_segment_sum(gate_embed, dst, zeros, n)
  return _tc_gru(
      sums[0], sums[1], cntp.T, node_states,
      W_g.T, b_g.reshape(1, hid),
      W_ih[:, :hid].T.astype(jnp.bfloat16),
      W_ih[:, hid:].T.astype(jnp.bfloat16),
      W_hh.T.astype(jnp.bfloat16),
      b_ih.reshape(1, 3 * hid), b_hh.reshape(1, 3 * hid),
      gamma.reshape(1, hid), beta.reshape(1, hid))


# speedup vs baseline: 1.1069x; 1.0013x over previous
import functools

import jax
import jax.numpy as jnp
from jax import lax
from jax.experimental import pallas as pl
from jax.experimental.pallas import tpu as pltpu
from jax.experimental.pallas import tpu_sc as plsc

_NS = 16
_NC = 2
_CH = 80


def _sc_segment_sum(gate_embed, dst_t, zeros, n_nodes):
  e_total, h = gate_embed.shape
  hh = h // _NC
  ept = e_total // _NS
  nchunks = ept // _CH
  ksub = _CH // 16
  nrchunk = -(-(-(-n_nodes // _NS)) // _CH)
  npt = nrchunk * _CH

  mesh = plsc.VectorSubcoreMesh(core_axis_name="c", subcore_axis_name="s")

  @functools.partial(
      pl.kernel,
      out_type=[
          jax.ShapeDtypeStruct((_NC, n_nodes, hh), jnp.float32),
          jax.ShapeDtypeStruct((_NS, n_nodes), jnp.float32),
      ],
      mesh=mesh,
      scratch_types=[
          pltpu.VMEM((nchunks, _CH), jnp.int32),
          pltpu.VMEM((_CH, hh), jnp.float32),
          pltpu.VMEM((_CH, hh), jnp.float32),
          pltpu.VMEM((n_nodes,), jnp.float32),
          pltpu.VMEM_SHARED((n_nodes, hh), jnp.float32),
          pltpu.SemaphoreType.DMA,
          pltpu.SemaphoreType.DMA,
      ],
      compiler_params=pltpu.CompilerParams(needs_layout_passes=False),
  )
  def k(gate_hbm, dst_hbm, zeros_hbm, sums_hbm, cntp_hbm,
        idx_v, gbuf, gbuf2, cnt_priv, acc, sem_a, sem_b):
    cid = lax.axis_index("c")
    sid = lax.axis_index("s")
    pltpu.sync_copy(zeros_hbm, gbuf)

    def zb(j, carry):
      row = sid * npt + j * _CH

      @pl.when(row + _CH <= n_nodes)
      def _():
        pltpu.sync_copy(gbuf, acc.at[pl.ds(row, _CH)])

      return carry

    lax.fori_loop(0, nrchunk, zb, 0)

    def zc(j, carry):
      cnt_priv[pl.ds(j * 16, 16)] = jnp.zeros((16,), jnp.float32)
      return carry

    lax.fori_loop(0, n_nodes // 16, zc, 0)
    pltpu.sync_copy(dst_hbm.at[sid], idx_v)
    plsc.subcore_barrier()
    ones16 = jnp.ones((16,), jnp.float32)

    def gate_src(g):
      row0 = sid * ept + g * _CH
      return gate_hbm.at[pl.ds(row0, _CH), pl.ds(cid * hh, hh)]

    def counts(g):
      @pl.when(cid == 0)
      def _():
        for kk in range(ksub):
          idx16 = idx_v[g, pl.ds(kk * 16, 16)]
          plsc.addupdate_scatter(cnt_priv, [idx16], ones16)

    dummy = gate_hbm.at[pl.ds(0, _CH), pl.ds(0, hh)]
    pltpu.async_copy(gate_src(0), gbuf, sem_a)

    def body(j, carry):
      g = j * 2
      pltpu.async_copy(gate_src(g + 1), gbuf2, sem_b)
      pltpu.make_async_copy(dummy, gbuf, sem_a).wait()
      pltpu.sync_copy(gbuf, acc.at[idx_v.at[g]], add=True)
      counts(g)
      pltpu.async_copy(gate_src(g + 2), gbuf, sem_a)
      pltpu.make_async_copy(dummy, gbuf2, sem_b).wait()
      pltpu.sync_copy(gbuf2, acc.at[idx_v.at[g + 1]], add=True)
      counts(g + 1)
      return carry

    lax.fori_loop(0, (nchunks - 1) // 2, body, 0)
    pltpu.make_async_copy(dummy, gbuf, sem_a).wait()
    pltpu.sync_copy(gbuf, acc.at[idx_v.at[nchunks - 1]], add=True)
    counts(nchunks - 1)
    plsc.subcore_barrier()

    def rb(j, carry):
      row = sid * npt + j * _CH

      @pl.when(row + _CH <= n_nodes)
      def _():
        pltpu.sync_copy(acc.at[pl.ds(row, _CH)], gbuf)
        pltpu.sync_copy(gbuf, sums_hbm.at[cid, pl.ds(row, _CH)])

      return carry

    lax.fori_loop(0, nrchunk, rb, 0)

    @pl.when(cid == 0)
    def _():
      pltpu.sync_copy(cnt_priv, cntp_hbm.at[sid])

  return k(gate_embed, dst_t, zeros)


def _row_spec(blk, w):
  return pl.BlockSpec((blk, w), lambda i: (i, 0))


def _full_spec(a, b):
  return pl.BlockSpec((a, b), lambda i: (0, 0))


def _tc_gru(s0, s1, cntp_t, h, wg_t, bg, wih_a, wih_b, whh_t, bih, bhh,
            gamma, beta):
  n, hid = h.shape
  blk = 2000
  grid = (n // blk,)

  def body(s0_r, s1_r, cnt_r, h_r, wg_r, bg_r, wa_r, wb_r, wh_r, bih_r,
           bhh_r, g_r, b_r, out_r):
    hh = h_r[...]
    hb = hh.astype(jnp.bfloat16)
    c = jnp.sum(cnt_r[...], axis=1, keepdims=True)
    s = jnp.concatenate([s0_r[...], s1_r[...]], axis=1)
    pre = jnp.dot(s, wg_r[...], preferred_element_type=jnp.float32)
    agg = (pre / jnp.maximum(c, 1.0)
           + jnp.where(c > 0.5, 1.0, 0.0) * bg_r[...])
    gi = (jnp.dot(agg.astype(jnp.bfloat16), wa_r[...],
                  preferred_element_type=jnp.float32)
          + jnp.dot(hb, wb_r[...], preferred_element_type=jnp.float32)
          + bih_r[...])
    gh = jnp.dot(hb, wh_r[...], preferred_element_type=jnp.float32) + bhh_r[...]
    r = jax.nn.sigmoid(gi[:, :hid] + gh[:, :hid])
    z = jax.nn.sigmoid(gi[:, hid:2 * hid] + gh[:, hid:2 * hid])
    nn = jnp.tanh(gi[:, 2 * hid:] + r * gh[:, 2 * hid:])
    new = (1.0 - z) * nn + z * hh
    mu = jnp.mean(new, axis=1, keepdims=True)
    var = jnp.mean((new - mu) ** 2, axis=1, keepdims=True)
    out_r[...] = (g_r[...] * (new - mu) * lax.rsqrt(var + 1e-5) + b_r[...])

  return pl.pallas_call(
      body,
      grid=grid,
      in_specs=[
          _row_spec(blk, s0.shape[1]),
          _row_spec(blk, s1.shape[1]),
          _row_spec(blk, cntp_t.shape[1]),
          _row_spec(blk, hid),
          _full_spec(*wg_t.shape),
          _full_spec(*bg.shape),
          _full_spec(*wih_a.shape),
          _full_spec(*wih_b.shape),
          _full_spec(*whh_t.shape),
          _full_spec(*bih.shape),
          _full_spec(*bhh.shape),
          _full_spec(*gamma.shape),
          _full_spec(*beta.shape),
      ],
      out_specs=_row_spec(blk, hid),
      out_shape=jax.ShapeDtypeStruct((n, hid), jnp.float32),
  )(s0, s1, cntp_t, h, wg_t, bg, wih_a, wih_b, whh_t, bih, bhh, gamma, beta)


def kernel(node_states, edge_index, gate_embed, n_nodes, W_g, b_g, W_ih,
           b_ih, W_hh, b_hh, gamma, beta):
  n, hid = node_states.shape
  e_total = gate_embed.shape[0]
  dst = edge_index[1].astype(jnp.int32).reshape(_NS, e_total // _NS // _CH,
                                                _CH)
  zeros = jnp.zeros((_CH, hid // _NC), jnp.float32)
  sums, cntp = _sc_segment_sum(gate_embed, dst, zeros, n)
  return _tc_gru(
      sums[0], sums[1], cntp.T, node_states,
      W_g.T, b_g.reshape(1, hid),
      W_ih[:, :hid].T.astype(jnp.bfloat16),
      W_ih[:, hid:].T.astype(jnp.bfloat16),
      W_hh.T.astype(jnp.bfloat16),
      b_ih.reshape(1, 3 * hid), b_hh.reshape(1, 3 * hid),
      gamma.reshape(1, hid), beta.reshape(1, hid))
